# Initial kernel scaffold; baseline (speedup 1.0000x reference)
import jax
import jax.numpy as jnp
from jax.experimental import pallas as pl


def kernel(x, edge_index, W1, b1, W2, b2, W3, b3):
    raise NotImplementedError("write your pallas kernel here")


# speedup vs baseline: 6.2779x; 6.2779x over previous
import functools

import jax
import jax.numpy as jnp
from jax import lax
from jax.experimental import pallas as pl
from jax.experimental.pallas import tpu as pltpu
from jax.experimental.pallas import tpu_sc as plsc

N = 10000
NP = 10240
E = 320000
D = 128

NC = 2
NS = 16
NW = NC * NS
EDGES_PER_TILE = E // NW
WIN = 80
NWIN = EDGES_PER_TILE // WIN
ROWS_PER_TILE = NP // NS


def _sc_segment_sum_body(h_hbm, src_hbm, dst_hbm, zeros_hbm, out_hbm,
                         src_v, dst_v, rows_v, agg_s, sem):
    c = lax.axis_index("c")
    s = lax.axis_index("s")
    wid = c * NS + s

    pltpu.sync_copy(zeros_hbm.at[pl.ds(s * ROWS_PER_TILE, ROWS_PER_TILE)],
                    agg_s.at[pl.ds(s * ROWS_PER_TILE, ROWS_PER_TILE)])

    pltpu.sync_copy(src_hbm.at[wid], src_v)
    pltpu.sync_copy(dst_hbm.at[wid], dst_v)

    plsc.subcore_barrier()

    def body(j, carry):
        pltpu.async_copy(h_hbm.at[src_v.at[j]], rows_v, sem).wait()
        pltpu.sync_copy(rows_v, agg_s.at[dst_v.at[j]], add=True)
        return carry

    lax.fori_loop(0, NWIN, body, 0)

    plsc.subcore_barrier()

    pltpu.sync_copy(agg_s.at[pl.ds(s * ROWS_PER_TILE, ROWS_PER_TILE)],
                    out_hbm.at[c, pl.ds(s * ROWS_PER_TILE, ROWS_PER_TILE)])


_sc_segment_sum = functools.partial(
    pl.kernel,
    mesh=plsc.VectorSubcoreMesh(core_axis_name="c", subcore_axis_name="s"),
    out_type=jax.ShapeDtypeStruct((NC, NP, D), jnp.float32),
    scratch_types=[
        pltpu.VMEM((NWIN, WIN), jnp.int32),
        pltpu.VMEM((NWIN, WIN), jnp.int32),
        pltpu.VMEM((WIN, D), jnp.float32),
        pltpu.VMEM_SHARED((NP, D), jnp.float32),
        pltpu.SemaphoreType.DMA,
    ],
)(_sc_segment_sum_body)


def _mm_body(h_ref, a0_ref, a1_ref, w_ref, b_ref, o_ref, *, act):
    acc = h_ref[...] + a0_ref[...] + a1_ref[...]
    y = jnp.dot(acc, w_ref[...], preferred_element_type=jnp.float32) + b_ref[...]
    if act:
        y = jnp.maximum(y, 0.0)
    o_ref[...] = y


def _tc_layer(h, a0, a1, w, b, act):
    blk = 1024
    grid = NP // blk
    return pl.pallas_call(
        functools.partial(_mm_body, act=act),
        grid=(grid,),
        in_specs=[
            pl.BlockSpec((blk, D), lambda i: (i, 0)),
            pl.BlockSpec((blk, D), lambda i: (i, 0)),
            pl.BlockSpec((blk, D), lambda i: (i, 0)),
            pl.BlockSpec((D, D), lambda i: (0, 0)),
            pl.BlockSpec((1, D), lambda i: (0, 0)),
        ],
        out_specs=pl.BlockSpec((blk, D), lambda i: (i, 0)),
        out_shape=jax.ShapeDtypeStruct((NP, D), jnp.float32),
    )(h, a0, a1, w, b.reshape(1, D))


def kernel(x, edge_index, W1, b1, W2, b2, W3, b3):
    ei = edge_index.astype(jnp.int32)
    src = ei[0].reshape(NW, NWIN, WIN)
    dst = ei[1].reshape(NW, NWIN, WIN)
    zeros = jnp.zeros((NP, D), jnp.float32)

    h = jnp.concatenate([x, jnp.zeros((NP - N, D), jnp.float32)], axis=0)
    for w, b, act in ((W1, b1, True), (W2, b2, True), (W3, b3, False)):
        parts = _sc_segment_sum(h, src, dst, zeros)
        h = _tc_layer(h, parts[0], parts[1], w, b, act)
    return h[:N]
